# baseline (speedup 1.0000x reference)
import functools

import jax
import jax.numpy as jnp
from jax import lax
from jax.experimental import pallas as pl
from jax.experimental.pallas import tpu as pltpu
from jax.experimental.pallas import tpu_sc as plsc

_N = 10000
_E = 320000
_P = 16
_NC = 2
_NS = 16
_NW = _NC * _NS
_WIN = 128
_EPT = _E // _NW
_SLO = 10240
_CPT = 10880
_WPT = _CPT // _WIN
_NPAD = 10240
_RPT = _NPAD // _NS
_NB = 5

_mesh = plsc.VectorSubcoreMesh(core_axis_name="c", subcore_axis_name="s")
_sc_params = pltpu.CompilerParams(use_tc_tiling_on_sc=False,
                                  needs_layout_passes=False,
                                  disable_bounds_checks=True)


def _fill(buf, start, count, value):
    v = jnp.full((16,), value, buf.dtype)

    def _f(i, carry):
        buf[pl.ds(start + i * 16, 16)] = v
        return carry

    lax.fori_loop(0, count // 16, _f, 0)


def _fast_rsqrt(d):
    i = plsc.bitcast(d, jnp.int32)
    i = jnp.full((16,), 0x5F3759DF, jnp.int32) - (i >> 1)
    y = plsc.bitcast(i, jnp.float32)
    for _ in range(3):
        y = y * (1.5 - 0.5 * d * y * y)
    return y


def _zero_acc(zbuf, acc, s):
    zero = jnp.zeros((_P,), jnp.float32)

    def _z(i, carry):
        zbuf[i, :] = zero
        return carry

    lax.fori_loop(0, _RPT, _z, 0)
    pltpu.sync_copy(zbuf, acc.at[pl.ds(s * _RPT, _RPT)])


def _stage_h(h_hbm, hstage, h_spm, s):
    pltpu.sync_copy(h_hbm.at[pl.ds(s * _RPT, _RPT)], hstage)
    pltpu.sync_copy(hstage, h_spm.at[pl.ds(s * _RPT, _RPT)])


def _msg_loop(rowbuf, colbuf, normbuf, h_spm, acc, rows_v, msg_v, sems):
    gbufs = [rows_v.at[i] for i in range(_NB)]
    mbufs = [msg_v.at[i] for i in range(_NB)]
    gsems = sems[:_NB]
    ssems = sems[_NB:]
    for i in range(_NB):
        pltpu.async_copy(h_spm.at[rowbuf.at[pl.ds(i * _WIN, _WIN)]],
                         gbufs[i], gsems[i])

    def _w(wg, carry):
        for par in range(_NB):
            gb, mb, gsem, ssem = gbufs[par], mbufs[par], gsems[par], ssems[par]
            w = wg * _NB + par
            e0 = w * _WIN
            pltpu.make_async_copy(
                h_spm.at[rowbuf.at[pl.ds(e0, _WIN)]], gb, gsem).wait()

            @pl.when(wg > 0)
            def _():
                pltpu.make_async_copy(
                    mb, acc.at[colbuf.at[pl.ds(e0 - _NB * _WIN, _WIN)]],
                    ssem).wait()

            def _m(k, carry2):
                n16 = normbuf[pl.ds(e0 + k * 16, 16)]
                for jj in range(16):
                    j = k * 16 + jj
                    mb[j, :] = gb[j, :] * n16[jj]
                return carry2

            lax.fori_loop(0, _WIN // 16, _m, 0)

            @pl.when(wg < _WPT // _NB - 1)
            def _():
                pltpu.async_copy(
                    h_spm.at[rowbuf.at[pl.ds(e0 + _NB * _WIN, _WIN)]],
                    gb, gsem)

            pltpu.async_copy(mb, acc.at[colbuf.at[pl.ds(e0, _WIN)]],
                             ssem, add=True)
        return carry

    lax.fori_loop(0, _WPT // _NB, _w, 0)
    for par in range(_NB):
        e0 = (_WPT - _NB + par) * _WIN
        pltpu.make_async_copy(
            mbufs[par], acc.at[colbuf.at[pl.ds(e0, _WIN)]], ssems[par]).wait()


_first_scratch = [
    pltpu.VMEM((_CPT,), jnp.int32),
    pltpu.VMEM((2 * _CPT,), jnp.int32),
    pltpu.VMEM((2 * _CPT,), jnp.float32),
    pltpu.VMEM((_NPAD,), jnp.float32),
    pltpu.VMEM((_RPT,), jnp.float32),
    pltpu.VMEM((_NB, _WIN, _P), jnp.float32),
    pltpu.VMEM((_NB, _WIN, _P), jnp.float32),
    pltpu.VMEM((_RPT, _P), jnp.float32),
    pltpu.VMEM((_RPT, _P), jnp.float32),
    pltpu.MemorySpace.VMEM_SHARED((_NPAD, _P), jnp.float32),
    pltpu.MemorySpace.VMEM_SHARED((_NPAD, _P), jnp.float32),
    pltpu.MemorySpace.VMEM_SHARED((_NPAD,), jnp.float32),
    [pltpu.SemaphoreType.DMA] * (2 * _NB),
]


@functools.partial(
    pl.kernel,
    out_type=(jax.ShapeDtypeStruct((_NC, _NPAD, _P), jnp.float32),
              jax.ShapeDtypeStruct((_NW * _CPT,), jnp.float32)),
    mesh=_mesh,
    compiler_params=_sc_params,
    scratch_types=_first_scratch,
)
def _agg_first(ei_hbm, ew_hbm, h_hbm,
               out_hbm, norm_out_hbm,
               rowbuf, ebuf, wbuf, degbuf, d2buf, rows_v, msg_v, zbuf,
               hstage, acc, h_spm, deg_spm, sems):
    c = lax.axis_index("c")
    s = lax.axis_index("s")
    wid = c * _NS + s
    mid = (1 - c) * _NS + s

    _zero_acc(zbuf, acc, s)
    _fill(d2buf, 0, _RPT, 0.0)
    pltpu.sync_copy(d2buf, deg_spm.at[pl.ds(s * _RPT, _RPT)])
    _stage_h(h_hbm, hstage, h_spm, s)

    pltpu.sync_copy(ei_hbm.at[0, pl.ds(wid * _EPT, _EPT)],
                    rowbuf.at[pl.ds(0, _EPT)])
    pltpu.sync_copy(ei_hbm.at[1, pl.ds(wid * _EPT, _EPT)],
                    ebuf.at[pl.ds(0, _EPT)])
    pltpu.sync_copy(ei_hbm.at[1, pl.ds(mid * _EPT, _EPT)],
                    ebuf.at[pl.ds(_CPT, _EPT)])
    pltpu.sync_copy(ew_hbm.at[pl.ds(wid * _EPT, _EPT)],
                    wbuf.at[pl.ds(0, _EPT)])
    pltpu.sync_copy(ew_hbm.at[pl.ds(mid * _EPT, _EPT)],
                    wbuf.at[pl.ds(_CPT, _EPT)])
    _fill(rowbuf, _EPT, _CPT - _EPT, 0)
    _fill(ebuf, _EPT, _CPT - _EPT, _NPAD - 1)
    _fill(ebuf, _CPT + _EPT, _CPT - _EPT, _NPAD - 1)
    _fill(wbuf, _EPT, _CPT - _EPT, 0.0)
    _fill(wbuf, _CPT + _EPT, _CPT - _EPT, 0.0)
    plsc.subcore_barrier()

    def _dw(w, carry):
        e0 = w * _WIN
        pltpu.async_copy(wbuf.at[pl.ds(e0, _WIN)],
                         deg_spm.at[ebuf.at[pl.ds(e0, _WIN)]],
                         sems[0], add=True)
        return carry

    lax.fori_loop(0, 2 * _WPT, _dw, 0)

    def _dd(w, carry):
        e0 = w * _WIN
        pltpu.make_async_copy(wbuf.at[pl.ds(e0, _WIN)],
                              deg_spm.at[ebuf.at[pl.ds(e0, _WIN)]],
                              sems[0]).wait()
        return carry

    lax.fori_loop(0, 2 * _WPT, _dd, 0)
    plsc.subcore_barrier()

    pltpu.sync_copy(deg_spm, degbuf)

    def _dv(i, carry):
        sl = pl.ds(i * 16, 16)
        degbuf[sl] = _fast_rsqrt(degbuf[sl] + 1.0)
        return carry

    lax.fori_loop(0, _NPAD // 16, _dv, 0)

    @pl.when(c == 0)
    def _():
        def _slw(i, carry):
            sl = pl.ds(_SLO + i * 16, 16)
            ids = s * _RPT + i * 16 + lax.iota(jnp.int32, 16)
            v = degbuf[pl.ds(s * _RPT + i * 16, 16)]
            rowbuf[sl] = ids
            ebuf[sl] = ids
            wbuf[sl] = v * v
            return carry

        lax.fori_loop(0, _RPT // 16, _slw, 0)

    def _nw(j, carry):
        sl = pl.ds(j * 16, 16)
        dr = plsc.load_gather(degbuf, [rowbuf[sl]])
        dc = plsc.load_gather(degbuf, [ebuf[sl]])
        wbuf[sl] = dr * wbuf[sl] * dc
        return carry

    lax.fori_loop(0, _SLO // 16, _nw, 0)
    pltpu.sync_copy(wbuf.at[pl.ds(0, _CPT)],
                    norm_out_hbm.at[pl.ds(wid * _CPT, _CPT)])

    _msg_loop(rowbuf, ebuf, wbuf, h_spm, acc, rows_v, msg_v, sems)
    plsc.subcore_barrier()
    pltpu.sync_copy(acc.at[pl.ds(s * _RPT, _RPT)],
                    out_hbm.at[c, pl.ds(s * _RPT, _RPT)])


_next_scratch = [
    pltpu.VMEM((_CPT,), jnp.int32),
    pltpu.VMEM((_CPT,), jnp.int32),
    pltpu.VMEM((_CPT,), jnp.float32),
    pltpu.VMEM((_NB, _WIN, _P), jnp.float32),
    pltpu.VMEM((_NB, _WIN, _P), jnp.float32),
    pltpu.VMEM((_RPT, _P), jnp.float32),
    pltpu.VMEM((_RPT, _P), jnp.float32),
    pltpu.MemorySpace.VMEM_SHARED((_NPAD, _P), jnp.float32),
    pltpu.MemorySpace.VMEM_SHARED((_NPAD, _P), jnp.float32),
    [pltpu.SemaphoreType.DMA] * (2 * _NB),
]


@functools.partial(
    pl.kernel,
    out_type=jax.ShapeDtypeStruct((_NC, _NPAD, _P), jnp.float32),
    mesh=_mesh,
    compiler_params=_sc_params,
    scratch_types=_next_scratch,
)
def _agg_next(ei_hbm, nrm_hbm, h_hbm, out_hbm,
              rowbuf, colbuf, normbuf, rows_v, msg_v, zbuf,
              hstage, acc, h_spm, sems):
    c = lax.axis_index("c")
    s = lax.axis_index("s")
    wid = c * _NS + s

    _zero_acc(zbuf, acc, s)
    _stage_h(h_hbm, hstage, h_spm, s)
    pltpu.sync_copy(ei_hbm.at[0, pl.ds(wid * _EPT, _EPT)],
                    rowbuf.at[pl.ds(0, _EPT)])
    pltpu.sync_copy(ei_hbm.at[1, pl.ds(wid * _EPT, _EPT)],
                    colbuf.at[pl.ds(0, _EPT)])
    pltpu.sync_copy(nrm_hbm.at[pl.ds(wid * _CPT, _CPT)], normbuf)
    _fill(rowbuf, _EPT, _CPT - _EPT, 0)
    _fill(colbuf, _EPT, _CPT - _EPT, _NPAD - 1)

    @pl.when(c == 0)
    def _():
        def _slw(i, carry):
            sl = pl.ds(_SLO + i * 16, 16)
            ids = s * _RPT + i * 16 + lax.iota(jnp.int32, 16)
            rowbuf[sl] = ids
            colbuf[sl] = ids
            return carry

        lax.fori_loop(0, _RPT // 16, _slw, 0)

    plsc.subcore_barrier()

    _msg_loop(rowbuf, colbuf, normbuf, h_spm, acc, rows_v, msg_v, sems)
    plsc.subcore_barrier()
    pltpu.sync_copy(acc.at[pl.ds(s * _RPT, _RPT)],
                    out_hbm.at[c, pl.ds(s * _RPT, _RPT)])


def _mm0(xP, K1):
    def body(x_ref, k_ref, o_ref):
        o_ref[:_N // 8, :] = jnp.dot(x_ref[...], k_ref[...],
                                     preferred_element_type=jnp.float32)
        o_ref[_N // 8:, :] = jnp.zeros(((_NPAD - _N) // 8, 128), jnp.float32)

    return pl.pallas_call(
        body,
        out_shape=jax.ShapeDtypeStruct((_NPAD // 8, 128), jnp.float32),
    )(xP, K1)


def _m12(p, b1t, K2):
    def body(p_ref, b_ref, k_ref, o_ref):
        t = jnp.maximum(p_ref[0] + p_ref[1] + b_ref[...], 0.0)
        o_ref[...] = jnp.dot(t, k_ref[...], preferred_element_type=jnp.float32)

    return pl.pallas_call(
        body,
        out_shape=jax.ShapeDtypeStruct((_NPAD // 8, 128), jnp.float32),
    )(p, b1t, K2)


def _e2(p, b2t):
    def body(p_ref, b_ref, o_ref):
        o_ref[...] = jnp.maximum(p_ref[0] + p_ref[1] + b_ref[...], 0.0)

    return pl.pallas_call(
        body,
        out_shape=jax.ShapeDtypeStruct((_NPAD // 8, 128), jnp.float32),
    )(p, b2t)


def _m3(p, K3, b3t):
    def body(p_ref, k_ref, b_ref, o_ref):
        agg = p_ref[0, :_N // 8, :] + p_ref[1, :_N // 8, :]
        o_ref[...] = jnp.dot(agg, k_ref[...],
                             preferred_element_type=jnp.float32) + b_ref[...]

    return pl.pallas_call(
        body,
        out_shape=jax.ShapeDtypeStruct((_N // 8, 8 * 128), jnp.float32),
    )(p, K3, b3t)


def kernel(x, edge_index, edge_weight, W1, b1, W2, b2, W3, b3):
    eye8 = jnp.eye(8, dtype=jnp.float32)
    W1p = jnp.pad(W1, ((0, 0), (0, _P - W1.shape[1])))
    K1 = jnp.kron(eye8, W1p)
    h0p = _mm0(x.reshape(_N // 8, 8 * 128), K1)

    p1, normp = _agg_first(edge_index, edge_weight,
                           h0p.reshape(_NPAD, _P))
    p1p = p1.reshape(_NC, _NPAD // 8, 128)

    K2 = jnp.kron(eye8, jnp.pad(W2, ((0, 1), (0, 1))))
    b1t = jnp.tile(jnp.pad(b1, (0, 1)), 8).reshape(1, 128)
    t1p = _m12(p1p, b1t, K2)

    p2 = _agg_next(edge_index, normp, t1p.reshape(_NPAD, _P))
    b2t = jnp.tile(jnp.pad(b2, (0, 1)), 8).reshape(1, 128)
    h2p = _e2(p2.reshape(_NC, _NPAD // 8, 128), b2t)

    p3 = _agg_next(edge_index, normp, h2p.reshape(_NPAD, _P))
    K3 = jnp.kron(eye8, jnp.pad(W3, ((0, 1), (0, 0))))
    b3t = jnp.tile(b3, 8).reshape(1, 8 * 128)
    outp = _m3(p3.reshape(_NC, _NPAD // 8, 128), K3, b3t)
    return outp.reshape(_N, 128)


# speedup vs baseline: 1.1060x; 1.1060x over previous
import functools

import jax
import jax.numpy as jnp
from jax import lax
from jax.experimental import pallas as pl
from jax.experimental.pallas import tpu as pltpu
from jax.experimental.pallas import tpu_sc as plsc

_N = 10000
_E = 320000
_P = 16
_NC = 2
_NS = 16
_NW = _NC * _NS
_WIN = 128
_EPT = _E // _NW
_CPT = 10240
_WPT = _CPT // _WIN
_NPAD = 10240
_RPT = _NPAD // _NS
_NB = 4

_mesh = plsc.VectorSubcoreMesh(core_axis_name="c", subcore_axis_name="s")
_sc_params = pltpu.CompilerParams(use_tc_tiling_on_sc=False,
                                  needs_layout_passes=False,
                                  disable_bounds_checks=True)


def _fill(buf, start, count, value):
    v = jnp.full((16,), value, buf.dtype)

    def _f(i, carry):
        buf[pl.ds(start + i * 16, 16)] = v
        return carry

    lax.fori_loop(0, count // 16, _f, 0)


def _fast_rsqrt(d):
    i = plsc.bitcast(d, jnp.int32)
    i = jnp.full((16,), 0x5F3759DF, jnp.int32) - (i >> 1)
    y = plsc.bitcast(i, jnp.float32)
    for _ in range(3):
        y = y * (1.5 - 0.5 * d * y * y)
    return y


def _zero_acc(zbuf, acc, s):
    zero = jnp.zeros((_P,), jnp.float32)

    def _z(i, carry):
        zbuf[i, :] = zero
        return carry

    lax.fori_loop(0, _RPT, _z, 0)
    pltpu.sync_copy(zbuf, acc.at[pl.ds(s * _RPT, _RPT)])


def _stage_h(h_hbm, hstage, h_spm, s):
    pltpu.sync_copy(h_hbm.at[pl.ds(s * _RPT, _RPT)], hstage)
    pltpu.sync_copy(hstage, h_spm.at[pl.ds(s * _RPT, _RPT)])


def _msg_loop(rowbuf, colbuf, normbuf, h_spm, acc, rows_v, msg_v, sems):
    gbufs = [rows_v.at[i] for i in range(_NB)]
    mbufs = [msg_v.at[i] for i in range(_NB)]
    gsems = sems[:_NB]
    ssems = sems[_NB:]
    for i in range(_NB):
        pltpu.async_copy(h_spm.at[rowbuf.at[pl.ds(i * _WIN, _WIN)]],
                         gbufs[i], gsems[i])

    def _w(wg, carry):
        for par in range(_NB):
            gb, mb, gsem, ssem = gbufs[par], mbufs[par], gsems[par], ssems[par]
            w = wg * _NB + par
            e0 = w * _WIN
            pltpu.make_async_copy(
                h_spm.at[rowbuf.at[pl.ds(e0, _WIN)]], gb, gsem).wait()

            @pl.when(wg > 0)
            def _():
                pltpu.make_async_copy(
                    mb, acc.at[colbuf.at[pl.ds(e0 - _NB * _WIN, _WIN)]],
                    ssem).wait()

            def _m(k, carry2):
                n16 = normbuf[pl.ds(e0 + k * 16, 16)]
                for jj in range(16):
                    j = k * 16 + jj
                    mb[j, :] = gb[j, :] * n16[jj]
                return carry2

            lax.fori_loop(0, _WIN // 16, _m, 0)

            @pl.when(wg < _WPT // _NB - 1)
            def _():
                pltpu.async_copy(
                    h_spm.at[rowbuf.at[pl.ds(e0 + _NB * _WIN, _WIN)]],
                    gb, gsem)

            pltpu.async_copy(mb, acc.at[colbuf.at[pl.ds(e0, _WIN)]],
                             ssem, add=True)
        return carry

    lax.fori_loop(0, _WPT // _NB, _w, 0)
    for par in range(_NB):
        e0 = (_WPT - _NB + par) * _WIN
        pltpu.make_async_copy(
            mbufs[par], acc.at[colbuf.at[pl.ds(e0, _WIN)]], ssems[par]).wait()


_first_scratch = [
    pltpu.VMEM((_CPT,), jnp.int32),
    pltpu.VMEM((2 * _CPT,), jnp.int32),
    pltpu.VMEM((2 * _CPT,), jnp.float32),
    pltpu.VMEM((_NPAD,), jnp.float32),
    pltpu.VMEM((_RPT,), jnp.float32),
    pltpu.VMEM((_NB, _WIN, _P), jnp.float32),
    pltpu.VMEM((_NB, _WIN, _P), jnp.float32),
    pltpu.VMEM((_RPT, _P), jnp.float32),
    pltpu.VMEM((_RPT, _P), jnp.float32),
    pltpu.MemorySpace.VMEM_SHARED((_NPAD, _P), jnp.float32),
    pltpu.MemorySpace.VMEM_SHARED((_NPAD, _P), jnp.float32),
    pltpu.MemorySpace.VMEM_SHARED((_NPAD,), jnp.float32),
    [pltpu.SemaphoreType.DMA] * (2 * _NB),
]


@functools.partial(
    pl.kernel,
    out_type=(jax.ShapeDtypeStruct((_NC, _NPAD, _P), jnp.float32),
              jax.ShapeDtypeStruct((_NW * _CPT,), jnp.float32),
              jax.ShapeDtypeStruct((_NPAD,), jnp.float32)),
    mesh=_mesh,
    compiler_params=_sc_params,
    scratch_types=_first_scratch,
)
def _agg_first(ei_hbm, ew_hbm, h_hbm,
               out_hbm, norm_out_hbm, d2_out_hbm,
               rowbuf, ebuf, wbuf, degbuf, d2buf, rows_v, msg_v, zbuf,
               hstage, acc, h_spm, deg_spm, sems):
    c = lax.axis_index("c")
    s = lax.axis_index("s")
    wid = c * _NS + s
    mid = (1 - c) * _NS + s

    _zero_acc(zbuf, acc, s)
    _fill(d2buf, 0, _RPT, 0.0)
    pltpu.sync_copy(d2buf, deg_spm.at[pl.ds(s * _RPT, _RPT)])
    _stage_h(h_hbm, hstage, h_spm, s)

    pltpu.sync_copy(ei_hbm.at[0, pl.ds(wid * _EPT, _EPT)],
                    rowbuf.at[pl.ds(0, _EPT)])
    pltpu.sync_copy(ei_hbm.at[1, pl.ds(wid * _EPT, _EPT)],
                    ebuf.at[pl.ds(0, _EPT)])
    pltpu.sync_copy(ei_hbm.at[1, pl.ds(mid * _EPT, _EPT)],
                    ebuf.at[pl.ds(_CPT, _EPT)])
    pltpu.sync_copy(ew_hbm.at[pl.ds(wid * _EPT, _EPT)],
                    wbuf.at[pl.ds(0, _EPT)])
    pltpu.sync_copy(ew_hbm.at[pl.ds(mid * _EPT, _EPT)],
                    wbuf.at[pl.ds(_CPT, _EPT)])
    _fill(rowbuf, _EPT, _CPT - _EPT, 0)
    _fill(ebuf, _EPT, _CPT - _EPT, _NPAD - 1)
    _fill(ebuf, _CPT + _EPT, _CPT - _EPT, _NPAD - 1)
    _fill(wbuf, _EPT, _CPT - _EPT, 0.0)
    _fill(wbuf, _CPT + _EPT, _CPT - _EPT, 0.0)
    plsc.subcore_barrier()

    def _dw(w, carry):
        e0 = w * _WIN
        pltpu.async_copy(wbuf.at[pl.ds(e0, _WIN)],
                         deg_spm.at[ebuf.at[pl.ds(e0, _WIN)]],
                         sems[0], add=True)
        return carry

    lax.fori_loop(0, 2 * _WPT, _dw, 0)

    def _dd(w, carry):
        e0 = w * _WIN
        pltpu.make_async_copy(wbuf.at[pl.ds(e0, _WIN)],
                              deg_spm.at[ebuf.at[pl.ds(e0, _WIN)]],
                              sems[0]).wait()
        return carry

    lax.fori_loop(0, 2 * _WPT, _dd, 0)
    plsc.subcore_barrier()

    pltpu.sync_copy(deg_spm, degbuf)

    def _dv(i, carry):
        sl = pl.ds(i * 16, 16)
        degbuf[sl] = _fast_rsqrt(degbuf[sl] + 1.0)
        return carry

    lax.fori_loop(0, _NPAD // 16, _dv, 0)

    @pl.when(c == 0)
    def _():
        def _d2(i, carry):
            sl = pl.ds(i * 16, 16)
            v = degbuf[pl.ds(s * _RPT + i * 16, 16)]
            d2buf[sl] = v * v
            return carry

        lax.fori_loop(0, _RPT // 16, _d2, 0)
        pltpu.sync_copy(d2buf, d2_out_hbm.at[pl.ds(s * _RPT, _RPT)])

    def _nw(j, carry):
        sl = pl.ds(j * 16, 16)
        dr = plsc.load_gather(degbuf, [rowbuf[sl]])
        dc = plsc.load_gather(degbuf, [ebuf[sl]])
        wbuf[sl] = dr * wbuf[sl] * dc
        return carry

    lax.fori_loop(0, _CPT // 16, _nw, 0)
    pltpu.sync_copy(wbuf.at[pl.ds(0, _CPT)],
                    norm_out_hbm.at[pl.ds(wid * _CPT, _CPT)])

    _msg_loop(rowbuf, ebuf, wbuf, h_spm, acc, rows_v, msg_v, sems)
    plsc.subcore_barrier()
    pltpu.sync_copy(acc.at[pl.ds(s * _RPT, _RPT)],
                    out_hbm.at[c, pl.ds(s * _RPT, _RPT)])


_next_scratch = [
    pltpu.VMEM((_CPT,), jnp.int32),
    pltpu.VMEM((_CPT,), jnp.int32),
    pltpu.VMEM((_CPT,), jnp.float32),
    pltpu.VMEM((_NB, _WIN, _P), jnp.float32),
    pltpu.VMEM((_NB, _WIN, _P), jnp.float32),
    pltpu.VMEM((_RPT, _P), jnp.float32),
    pltpu.VMEM((_RPT, _P), jnp.float32),
    pltpu.MemorySpace.VMEM_SHARED((_NPAD, _P), jnp.float32),
    pltpu.MemorySpace.VMEM_SHARED((_NPAD, _P), jnp.float32),
    [pltpu.SemaphoreType.DMA] * (2 * _NB),
]


@functools.partial(
    pl.kernel,
    out_type=jax.ShapeDtypeStruct((_NC, _NPAD, _P), jnp.float32),
    mesh=_mesh,
    compiler_params=_sc_params,
    scratch_types=_next_scratch,
)
def _agg_next(ei_hbm, nrm_hbm, h_hbm, out_hbm,
              rowbuf, colbuf, normbuf, rows_v, msg_v, zbuf,
              hstage, acc, h_spm, sems):
    c = lax.axis_index("c")
    s = lax.axis_index("s")
    wid = c * _NS + s

    _zero_acc(zbuf, acc, s)
    _stage_h(h_hbm, hstage, h_spm, s)
    pltpu.sync_copy(ei_hbm.at[0, pl.ds(wid * _EPT, _EPT)],
                    rowbuf.at[pl.ds(0, _EPT)])
    pltpu.sync_copy(ei_hbm.at[1, pl.ds(wid * _EPT, _EPT)],
                    colbuf.at[pl.ds(0, _EPT)])
    pltpu.sync_copy(nrm_hbm.at[pl.ds(wid * _CPT, _CPT)], normbuf)
    _fill(rowbuf, _EPT, _CPT - _EPT, 0)
    _fill(colbuf, _EPT, _CPT - _EPT, _NPAD - 1)
    plsc.subcore_barrier()

    _msg_loop(rowbuf, colbuf, normbuf, h_spm, acc, rows_v, msg_v, sems)
    plsc.subcore_barrier()
    pltpu.sync_copy(acc.at[pl.ds(s * _RPT, _RPT)],
                    out_hbm.at[c, pl.ds(s * _RPT, _RPT)])


def _mm0(x, W1p):
    def body(x_ref, w_ref, o_ref):
        o_ref[:_N, :] = jnp.dot(x_ref[...], w_ref[...],
                                preferred_element_type=jnp.float32)
        o_ref[_N:, :] = jnp.zeros((_NPAD - _N, _P), jnp.float32)

    return pl.pallas_call(
        body,
        out_shape=jax.ShapeDtypeStruct((_NPAD, _P), jnp.float32),
    )(x, W1p)


def _m12(p, d2, h0, b1p, W2p):
    def body(p_ref, d2_ref, h_ref, b_ref, w_ref, o_ref):
        agg = p_ref[0] + p_ref[1] + d2_ref[...] * h_ref[...]
        t = jnp.maximum(agg + b_ref[...], 0.0)
        o_ref[...] = jnp.dot(t, w_ref[...], preferred_element_type=jnp.float32)

    return pl.pallas_call(
        body,
        out_shape=jax.ShapeDtypeStruct((_NPAD, _P), jnp.float32),
    )(p, d2, h0, b1p, W2p)


def _e2(p, d2, t1, b2p):
    def body(p_ref, d2_ref, h_ref, b_ref, o_ref):
        agg = p_ref[0] + p_ref[1] + d2_ref[...] * h_ref[...]
        o_ref[...] = jnp.maximum(agg + b_ref[...], 0.0)

    return pl.pallas_call(
        body,
        out_shape=jax.ShapeDtypeStruct((_NPAD, _P), jnp.float32),
    )(p, d2, t1, b2p)


def _m3(p, d2, h2, W3p, b3p):
    def body(p_ref, d2_ref, h_ref, w_ref, b_ref, o_ref):
        agg = (p_ref[0, :_N, :] + p_ref[1, :_N, :]
               + d2_ref[:_N, :] * h_ref[:_N, :])
        o_ref[...] = jnp.dot(agg, w_ref[...],
                             preferred_element_type=jnp.float32) + b_ref[...]

    return pl.pallas_call(
        body,
        out_shape=jax.ShapeDtypeStruct((_N, 128), jnp.float32),
    )(p, d2, h2, W3p, b3p)


def kernel(x, edge_index, edge_weight, W1, b1, W2, b2, W3, b3):
    W1p = jnp.pad(W1, ((0, 0), (0, _P - W1.shape[1])))
    h0 = _mm0(x, W1p)

    p1, normp, d2f = _agg_first(edge_index, edge_weight, h0)
    d2 = d2f.reshape(_NPAD, 1)

    W2p = jnp.pad(W2, ((0, 1), (0, 1)))
    b1p = jnp.pad(b1, (0, 1)).reshape(1, _P)
    t1 = _m12(p1, d2, h0, b1p, W2p)

    p2 = _agg_next(edge_index, normp, t1)
    b2p = jnp.pad(b2, (0, 1)).reshape(1, _P)
    h2 = _e2(p2, d2, t1, b2p)

    p3 = _agg_next(edge_index, normp, h2)
    W3p = jnp.pad(W3, ((0, 1), (0, 0)))
    return _m3(p3, d2, h2, W3p, b3.reshape(1, 128))
